# baseline (speedup 1.0000x reference)
import functools

import jax
import jax.numpy as jnp
from jax import lax
from jax.experimental import pallas as pl
from jax.experimental.pallas import tpu as pltpu
from jax.experimental.pallas import tpu_sc as plsc

B = 16384
NUM_FEAT = 32
CAT_VOCAB = 1001
CAT_DIM = 16
EMB_OTHER = 64
HIDDEN = 128


def _sc_pair_gather(i0, i1, t0, t1, width, transform):
    info = plsc.get_sparse_core_info()
    NC, NS = info.num_cores, info.num_subcores
    NW = NC * NS
    n = B // NW
    CH = 64
    nchunk = n // CH

    mesh = plsc.VectorSubcoreMesh(core_axis_name="c", subcore_axis_name="s")

    @functools.partial(
        pl.kernel,
        mesh=mesh,
        out_type=[
            jax.ShapeDtypeStruct((B, width), jnp.float32),
            jax.ShapeDtypeStruct((B, width), jnp.float32),
        ],
        scratch_types=[
            pltpu.VMEM((2, B // (2 * 16)), jnp.int32),
            pltpu.VMEM((2, CH, width), jnp.float32),
            pltpu.VMEM((2, CH, width), jnp.float32),
            pltpu.SemaphoreType.DMA,
        ],
    )
    def k(i0_h, i1_h, t0_h, t1_h, a_o, b_o, idx_v, a_b, b_b, sem):
        wid = lax.axis_index("s") * NC + lax.axis_index("c")
        base = wid * n
        src = pl.ds(base, n)
        pltpu.sync_copy(i0_h.at[src], idx_v.at[0])
        pltpu.sync_copy(i1_h.at[src], idx_v.at[1])

        def drain_and_flush(c):
            s = c % 2
            pltpu.make_async_copy(t0_h.at[pl.ds(0, CH), :], a_b.at[s], sem).wait()
            pltpu.make_async_copy(t1_h.at[pl.ds(0, CH), :], b_b.at[s], sem).wait()
            out = pl.ds(base + c * CH, CH)
            pltpu.sync_copy(a_b.at[s], a_o.at[out])
            pltpu.sync_copy(b_b.at[s], b_o.at[out])

        for c in range(nchunk):
            s = c % 2

            def issue_group(g, _):
                qb = c * CH + g * 16
                v0 = transform(idx_v[0, pl.ds(qb, 16)])
                v1 = transform(idx_v[1, pl.ds(qb, 16)])
                for lane in range(16):
                    row = pl.ds(g * 16 + lane, 1)
                    pltpu.async_copy(t0_h.at[pl.ds(v0[lane], 1), :], a_b.at[s, row, :], sem)
                    pltpu.async_copy(t1_h.at[pl.ds(v1[lane], 1), :], b_b.at[s, row, :], sem)
                return _

            lax.fori_loop(0, CH // 16, issue_group, None)
            if c > 0:
                drain_and_flush(c - 1)
        drain_and_flush(nchunk - 1)

    return k(i0, i1, t0, t1)


_BLK = 1024


def _trans_body(xn, e0r, e1r, wt, bt, to):
    acc = jnp.dot(xn[:], wt[pl.ds(0, NUM_FEAT), :],
                  preferred_element_type=jnp.float32)
    acc += jnp.dot(e0r[:], wt[pl.ds(NUM_FEAT, CAT_DIM), :],
                   preferred_element_type=jnp.float32)
    acc += jnp.dot(e1r[:], wt[pl.ds(NUM_FEAT + CAT_DIM, CAT_DIM), :],
                   preferred_element_type=jnp.float32)
    to[:] = acc + bt[:]


def _cm_body(cr, mr, wc, bc, wm, bm, co, mo):
    co[:] = jnp.dot(cr[:], wc[:], preferred_element_type=jnp.float32) + bc[:]
    mo[:] = jnp.dot(mr[:], wm[:], preferred_element_type=jnp.float32) + bm[:]


def _row_blk(w):
    return pl.BlockSpec((_BLK, w), lambda i: (i, 0))


def _full(a):
    return pl.BlockSpec(a.shape, lambda i: (0,) * a.ndim)


def _tc_trans(x_num, e0, e1, W_trans, b_trans):
    return pl.pallas_call(
        _trans_body,
        grid=(B // _BLK,),
        in_specs=[_row_blk(NUM_FEAT), _row_blk(CAT_DIM), _row_blk(CAT_DIM),
                  _full(W_trans), _full(b_trans)],
        out_specs=[_row_blk(HIDDEN)],
        out_shape=[jax.ShapeDtypeStruct((B, HIDDEN), jnp.float32)],
    )(x_num, e0, e1, W_trans, b_trans)[0]


def _tc_cm(card_rows, merch_rows, W_card, b_card, W_merch, b_merch):
    return pl.pallas_call(
        _cm_body,
        grid=(B // _BLK,),
        in_specs=[_row_blk(EMB_OTHER), _row_blk(EMB_OTHER),
                  _full(W_card), _full(b_card), _full(W_merch), _full(b_merch)],
        out_specs=[_row_blk(HIDDEN), _row_blk(HIDDEN)],
        out_shape=[jax.ShapeDtypeStruct((B, HIDDEN), jnp.float32)] * 2,
    )(card_rows, merch_rows, W_card, b_card, W_merch, b_merch)


def kernel(x_num, x_cat, n_id_card, n_id_merchant,
           emb_pcd, emb_ct, W_trans, b_trans,
           emb_card, W_card, b_card,
           emb_merch, W_merch, b_merch):
    xc0 = x_cat[:, 0].astype(jnp.int32)
    xc1 = x_cat[:, 1].astype(jnp.int32)
    clip = lambda v: jnp.clip(v + 1, 0, CAT_VOCAB - 1)
    ident = lambda v: v
    e0, e1 = _sc_pair_gather(xc0, xc1, emb_pcd, emb_ct, CAT_DIM, clip)
    card_rows, merch_rows = _sc_pair_gather(
        n_id_card.astype(jnp.int32), n_id_merchant.astype(jnp.int32),
        emb_card, emb_merch, EMB_OTHER, ident)
    trans_out = _tc_trans(x_num, e0, e1, W_trans, b_trans.reshape(1, HIDDEN))
    card_out, merch_out = _tc_cm(card_rows, merch_rows,
                                 W_card, b_card.reshape(1, HIDDEN),
                                 W_merch, b_merch.reshape(1, HIDDEN))
    return (trans_out, card_out, merch_out)
